# Initial kernel scaffold; baseline (speedup 1.0000x reference)
import jax
import jax.numpy as jnp
from jax.experimental import pallas as pl


def kernel(smpl_v_orig, object_v_orig, rel_dist, smpl_anchor_indices, object_anchor_indices):
    raise NotImplementedError("write your pallas kernel here")


# speedup vs baseline: 1.5872x; 1.5872x over previous
import functools

import jax
import jax.numpy as jnp
from jax import lax
from jax.experimental import pallas as pl
from jax.experimental.pallas import tpu as pltpu
from jax.experimental.pallas import tpu_sc as plsc

_NC, _NS = 2, 16
_CHUNK = 128
_ROW = 8


def _sc_gather(smpl_tab, obj_tab, sidx2, oidx3):
    B = smpl_tab.shape[0]
    ns_chunks, nc = sidx2.shape
    no_chunks = oidx3.shape[1]
    A_S = ns_chunks * nc
    A_O = no_chunks * nc

    mesh = plsc.VectorSubcoreMesh(core_axis_name="c", subcore_axis_name="s")

    @functools.partial(
        pl.kernel,
        out_type=(
            jax.ShapeDtypeStruct((B, A_S, _ROW), jnp.float32),
            jax.ShapeDtypeStruct((B, A_O, _ROW), jnp.float32),
        ),
        mesh=mesh,
        scratch_types=[
            pltpu.VMEM((ns_chunks, nc), jnp.int32),
            pltpu.VMEM((no_chunks, nc), jnp.int32),
            pltpu.VMEM((A_S, _ROW), jnp.float32),
            pltpu.VMEM((A_O, _ROW), jnp.float32),
            pltpu.SemaphoreType.DMA,
        ],
        compiler_params=pltpu.CompilerParams(use_tc_tiling_on_sc=False),
    )
    def gather_kernel(smpl_hbm, obj_hbm, sidx_hbm, oidx_hbm,
                      out_s_hbm, out_o_hbm,
                      sidx_vm, oidx_vm, srows_vm, orows_vm, sem):
        b = lax.axis_index("s") * _NC + lax.axis_index("c")
        pltpu.sync_copy(sidx_hbm, sidx_vm)
        pltpu.sync_copy(oidx_hbm.at[b], oidx_vm)

        copies = []
        for j in range(ns_chunks):
            copies.append(pltpu.async_copy(
                smpl_hbm.at[b].at[sidx_vm.at[j]],
                srows_vm.at[pl.ds(j * nc, nc)], sem))
        for j in range(no_chunks):
            copies.append(pltpu.async_copy(
                obj_hbm.at[b].at[oidx_vm.at[j]],
                orows_vm.at[pl.ds(j * nc, nc)], sem))
        for c in copies:
            c.wait()

        pltpu.sync_copy(srows_vm, out_s_hbm.at[b])
        pltpu.sync_copy(orows_vm, out_o_hbm.at[b])

    return gather_kernel(smpl_tab, obj_tab, sidx2, oidx3)


def _tc_loss_sum(rel3, smpl_rows, obj_g3, tile):
    B, A_S, L3 = rel3.shape
    nt = A_S // tile

    def body(rel_ref, smpl_ref, obj_ref, out_ref):
        step = pl.program_id(0) * nt + pl.program_id(1)
        rel = rel_ref[0]
        smpl = smpl_ref[0]
        obj = obj_ref[0]
        lane = lax.broadcasted_iota(jnp.int32, (1, L3), 1)
        lane_c = lane - (lane // 3) * 3
        acc = obj - rel
        for c in range(3):
            mask = (lane_c == c).astype(jnp.float32)
            acc = acc - smpl[:, c:c + 1] * mask
        psum = jnp.sum(jnp.abs(acc))

        @pl.when(step == 0)
        def _():
            out_ref[0, 0] = psum

        @pl.when(step != 0)
        def _():
            out_ref[0, 0] += psum

    return pl.pallas_call(
        body,
        grid=(B, nt),
        in_specs=[
            pl.BlockSpec((1, tile, L3), lambda b, i: (b, i, 0)),
            pl.BlockSpec((1, tile, _ROW), lambda b, i: (b, i, 0)),
            pl.BlockSpec((1, 1, L3), lambda b, i: (b, 0, 0)),
        ],
        out_specs=pl.BlockSpec(memory_space=pltpu.SMEM),
        out_shape=jax.ShapeDtypeStruct((1, 1), jnp.float32),
        compiler_params=pltpu.CompilerParams(
            dimension_semantics=("arbitrary", "arbitrary")),
    )(rel3, smpl_rows, obj_g3)


def kernel(smpl_v_orig, object_v_orig, rel_dist, smpl_anchor_indices,
           object_anchor_indices):
    B, n_smpl, _ = smpl_v_orig.shape
    n_obj = object_v_orig.shape[1]
    A_S = smpl_anchor_indices.shape[0]
    A_O = object_anchor_indices.shape[1]

    pad = ((0, 0), (0, 0), (0, _ROW - 3))
    smpl_tab = jnp.pad(smpl_v_orig, pad)
    obj_tab = jnp.pad(object_v_orig, pad)

    sidx2 = smpl_anchor_indices.astype(jnp.int32).reshape(
        A_S // _CHUNK, _CHUNK)
    oidx3 = object_anchor_indices.astype(jnp.int32).reshape(
        B, A_O // _CHUNK, _CHUNK)

    smpl_rows, obj_rows = _sc_gather(smpl_tab, obj_tab, sidx2, oidx3)

    rel3 = rel_dist.reshape(B, A_S, A_O * 3)
    obj_g3 = obj_rows[:, :, :3].reshape(B, 1, A_O * 3)

    total = _tc_loss_sum(rel3, smpl_rows, obj_g3, tile=256)
    count = B * A_S * A_O * 3
    return (total / count).reshape(())


# speedup vs baseline: 1.6704x; 1.0524x over previous
import functools

import jax
import jax.numpy as jnp
from jax import lax
from jax.experimental import pallas as pl
from jax.experimental.pallas import tpu as pltpu
from jax.experimental.pallas import tpu_sc as plsc

_NC, _NS = 2, 16
_CHUNK = 128
_ROW = 8


def _sc_gather(smpl_tab, obj_tab, sidx2, oidx3):
    B = smpl_tab.shape[0]
    ns_chunks, nc = sidx2.shape
    no_chunks = oidx3.shape[1]
    A_S = ns_chunks * nc
    A_O = no_chunks * nc

    mesh = plsc.VectorSubcoreMesh(core_axis_name="c", subcore_axis_name="s")

    @functools.partial(
        pl.kernel,
        out_type=(
            jax.ShapeDtypeStruct((B, A_S, _ROW), jnp.float32),
            jax.ShapeDtypeStruct((B, A_O, _ROW), jnp.float32),
        ),
        mesh=mesh,
        scratch_types=[
            pltpu.VMEM((ns_chunks, nc), jnp.int32),
            pltpu.VMEM((no_chunks, nc), jnp.int32),
            pltpu.VMEM((A_S, _ROW), jnp.float32),
            pltpu.VMEM((A_O, _ROW), jnp.float32),
            pltpu.SemaphoreType.DMA,
        ],
        compiler_params=pltpu.CompilerParams(use_tc_tiling_on_sc=False),
    )
    def gather_kernel(smpl_hbm, obj_hbm, sidx_hbm, oidx_hbm,
                      out_s_hbm, out_o_hbm,
                      sidx_vm, oidx_vm, srows_vm, orows_vm, sem):
        b = lax.axis_index("s") * _NC + lax.axis_index("c")
        pltpu.sync_copy(sidx_hbm, sidx_vm)
        pltpu.sync_copy(oidx_hbm.at[b], oidx_vm)

        copies = []
        for j in range(ns_chunks):
            copies.append(pltpu.async_copy(
                smpl_hbm.at[b].at[sidx_vm.at[j]],
                srows_vm.at[pl.ds(j * nc, nc)], sem))
        for j in range(no_chunks):
            copies.append(pltpu.async_copy(
                obj_hbm.at[b].at[oidx_vm.at[j]],
                orows_vm.at[pl.ds(j * nc, nc)], sem))
        for c in copies:
            c.wait()

        pltpu.sync_copy(srows_vm, out_s_hbm.at[b])
        pltpu.sync_copy(orows_vm, out_o_hbm.at[b])

    return gather_kernel(smpl_tab, obj_tab, sidx2, oidx3)


def _tc_loss_sum(rel3, smpl_rows, obj_g3, tile):
    B, A_S, L3 = rel3.shape
    nt = A_S // tile

    def body(rel_ref, smpl_ref, obj_ref, out_ref):
        step = pl.program_id(0) * nt + pl.program_id(1)
        rel = rel_ref[0]
        smpl = smpl_ref[0]
        obj = obj_ref[0]
        lane = lax.broadcasted_iota(jnp.int32, (1, L3), 1)
        lane_c = lane - (lane // 3) * 3
        s0 = smpl[:, 0:1]
        s1 = smpl[:, 1:2]
        s2 = smpl[:, 2:3]
        smpl_bc = jnp.where(lane_c == 0, s0, jnp.where(lane_c == 1, s1, s2))
        absdiff = jnp.abs((obj - smpl_bc) - rel)
        ones = jnp.ones((L3, 1), jnp.float32)
        psum = jnp.sum(jax.lax.dot_general(
            absdiff, ones, (((1,), (0,)), ((), ())),
            preferred_element_type=jnp.float32))

        @pl.when(step == 0)
        def _():
            out_ref[0, 0] = psum

        @pl.when(step != 0)
        def _():
            out_ref[0, 0] += psum

    return pl.pallas_call(
        body,
        grid=(B, nt),
        in_specs=[
            pl.BlockSpec((1, tile, L3), lambda b, i: (b, i, 0)),
            pl.BlockSpec((1, tile, _ROW), lambda b, i: (b, i, 0)),
            pl.BlockSpec((1, 1, L3), lambda b, i: (b, 0, 0)),
        ],
        out_specs=pl.BlockSpec(memory_space=pltpu.SMEM),
        out_shape=jax.ShapeDtypeStruct((1, 1), jnp.float32),
        compiler_params=pltpu.CompilerParams(
            dimension_semantics=("arbitrary", "arbitrary")),
    )(rel3, smpl_rows, obj_g3)


def kernel(smpl_v_orig, object_v_orig, rel_dist, smpl_anchor_indices,
           object_anchor_indices):
    B, n_smpl, _ = smpl_v_orig.shape
    n_obj = object_v_orig.shape[1]
    A_S = smpl_anchor_indices.shape[0]
    A_O = object_anchor_indices.shape[1]

    pad = ((0, 0), (0, 0), (0, _ROW - 3))
    smpl_tab = jnp.pad(smpl_v_orig, pad)
    obj_tab = jnp.pad(object_v_orig, pad)

    sidx2 = smpl_anchor_indices.astype(jnp.int32).reshape(
        A_S // _CHUNK, _CHUNK)
    oidx3 = object_anchor_indices.astype(jnp.int32).reshape(
        B, A_O // _CHUNK, _CHUNK)

    smpl_rows, obj_rows = _sc_gather(smpl_tab, obj_tab, sidx2, oidx3)

    rel3 = rel_dist.reshape(B, A_S, A_O * 3)
    obj_g3 = obj_rows[:, :, :3].reshape(B, 1, A_O * 3)

    total = _tc_loss_sum(rel3, smpl_rows, obj_g3, tile=512)
    count = B * A_S * A_O * 3
    return (total / count).reshape(())


# speedup vs baseline: 1.6705x; 1.0000x over previous
import functools

import jax
import jax.numpy as jnp
from jax import lax
from jax.experimental import pallas as pl
from jax.experimental.pallas import tpu as pltpu
from jax.experimental.pallas import tpu_sc as plsc

_NC, _NS = 2, 16
_CHUNK = 128
_ROW = 8


def _sc_gather(smpl_tab, obj_tab, sidx2, oidx3):
    B = smpl_tab.shape[0]
    ns_chunks, nc = sidx2.shape
    no_chunks = oidx3.shape[1]
    A_S = ns_chunks * nc
    A_O = no_chunks * nc

    mesh = plsc.VectorSubcoreMesh(core_axis_name="c", subcore_axis_name="s")

    @functools.partial(
        pl.kernel,
        out_type=(
            jax.ShapeDtypeStruct((B, A_S, _ROW), jnp.float32),
            jax.ShapeDtypeStruct((B, A_O, _ROW), jnp.float32),
        ),
        mesh=mesh,
        scratch_types=[
            pltpu.VMEM((ns_chunks, nc), jnp.int32),
            pltpu.VMEM((no_chunks, nc), jnp.int32),
            pltpu.VMEM((A_S, _ROW), jnp.float32),
            pltpu.VMEM((A_O, _ROW), jnp.float32),
            pltpu.SemaphoreType.DMA,
        ],
        compiler_params=pltpu.CompilerParams(use_tc_tiling_on_sc=False),
    )
    def gather_kernel(smpl_hbm, obj_hbm, sidx_hbm, oidx_hbm,
                      out_s_hbm, out_o_hbm,
                      sidx_vm, oidx_vm, srows_vm, orows_vm, sem):
        b = lax.axis_index("s") * _NC + lax.axis_index("c")
        pltpu.sync_copy(sidx_hbm, sidx_vm)
        pltpu.sync_copy(oidx_hbm.at[b], oidx_vm)

        copies = []
        for j in range(ns_chunks):
            copies.append(pltpu.async_copy(
                smpl_hbm.at[b].at[sidx_vm.at[j]],
                srows_vm.at[pl.ds(j * nc, nc)], sem))
        for j in range(no_chunks):
            copies.append(pltpu.async_copy(
                obj_hbm.at[b].at[oidx_vm.at[j]],
                orows_vm.at[pl.ds(j * nc, nc)], sem))
        for c in copies:
            c.wait()

        pltpu.sync_copy(srows_vm, out_s_hbm.at[b])
        pltpu.sync_copy(orows_vm, out_o_hbm.at[b])

    return gather_kernel(smpl_tab, obj_tab, sidx2, oidx3)


def _tc_loss_sum(rel3, smpl_rows, obj_g3, tile):
    B, A_S, L3 = rel3.shape
    nt = A_S // tile

    def body(rel_ref, smpl_ref, obj_ref, out_ref):
        step = pl.program_id(0) * nt + pl.program_id(1)
        rel = rel_ref[0]
        smpl = smpl_ref[0]
        obj = obj_ref[0]
        col = lax.broadcasted_iota(jnp.int32, (tile, 8), 1)
        smpl_aug = jnp.where(col == 3, 1.0, smpl)
        lane = lax.broadcasted_iota(jnp.int32, (8, L3), 1)
        row = lax.broadcasted_iota(jnp.int32, (8, L3), 0)
        lane_c = lane - (lane // 3) * 3
        m8 = jnp.where(row == 3, obj, jnp.where(row == lane_c, -1.0, 0.0))
        base = lax.dot_general(
            smpl_aug, m8, (((1,), (0,)), ((), ())),
            preferred_element_type=jnp.float32)
        absdiff = jnp.abs(base - rel)
        ones = jnp.ones((L3, 1), jnp.float32)
        psum = jnp.sum(jax.lax.dot_general(
            absdiff, ones, (((1,), (0,)), ((), ())),
            preferred_element_type=jnp.float32))

        @pl.when(step == 0)
        def _():
            out_ref[0, 0] = psum

        @pl.when(step != 0)
        def _():
            out_ref[0, 0] += psum

    return pl.pallas_call(
        body,
        grid=(B, nt),
        in_specs=[
            pl.BlockSpec((1, tile, L3), lambda b, i: (b, i, 0)),
            pl.BlockSpec((1, tile, _ROW), lambda b, i: (b, i, 0)),
            pl.BlockSpec((1, 1, L3), lambda b, i: (b, 0, 0)),
        ],
        out_specs=pl.BlockSpec(memory_space=pltpu.SMEM),
        out_shape=jax.ShapeDtypeStruct((1, 1), jnp.float32),
        compiler_params=pltpu.CompilerParams(
            dimension_semantics=("arbitrary", "arbitrary")),
    )(rel3, smpl_rows, obj_g3)


def kernel(smpl_v_orig, object_v_orig, rel_dist, smpl_anchor_indices,
           object_anchor_indices):
    B, n_smpl, _ = smpl_v_orig.shape
    n_obj = object_v_orig.shape[1]
    A_S = smpl_anchor_indices.shape[0]
    A_O = object_anchor_indices.shape[1]

    pad = ((0, 0), (0, 0), (0, _ROW - 3))
    smpl_tab = jnp.pad(smpl_v_orig, pad)
    obj_tab = jnp.pad(object_v_orig, pad)

    sidx2 = smpl_anchor_indices.astype(jnp.int32).reshape(
        A_S // _CHUNK, _CHUNK)
    oidx3 = object_anchor_indices.astype(jnp.int32).reshape(
        B, A_O // _CHUNK, _CHUNK)

    smpl_rows, obj_rows = _sc_gather(smpl_tab, obj_tab, sidx2, oidx3)

    rel3 = rel_dist.reshape(B, A_S, A_O * 3)
    obj_g3 = obj_rows[:, :, :3].reshape(B, 1, A_O * 3)

    total = _tc_loss_sum(rel3, smpl_rows, obj_g3, tile=512)
    count = B * A_S * A_O * 3
    return (total / count).reshape(())


# speedup vs baseline: 1.7199x; 1.0296x over previous
import functools

import jax
import jax.numpy as jnp
from jax import lax
from jax.experimental import pallas as pl
from jax.experimental.pallas import tpu as pltpu
from jax.experimental.pallas import tpu_sc as plsc

_NC, _NS = 2, 16
_CHUNK = 128
_ROW = 8


def _sc_gather(smpl_tab, obj_tab, sidx2, oidx3):
    B = smpl_tab.shape[0]
    ns_chunks, nc = sidx2.shape
    no_chunks = oidx3.shape[1]
    A_S = ns_chunks * nc
    A_O = no_chunks * nc

    mesh = plsc.VectorSubcoreMesh(core_axis_name="c", subcore_axis_name="s")

    @functools.partial(
        pl.kernel,
        out_type=(
            jax.ShapeDtypeStruct((B, A_S, _ROW), jnp.float32),
            jax.ShapeDtypeStruct((B, A_O, _ROW), jnp.float32),
        ),
        mesh=mesh,
        scratch_types=[
            pltpu.VMEM((ns_chunks, nc), jnp.int32),
            pltpu.VMEM((no_chunks, nc), jnp.int32),
            pltpu.VMEM((A_S, _ROW), jnp.float32),
            pltpu.VMEM((A_O, _ROW), jnp.float32),
            pltpu.SemaphoreType.DMA,
        ],
        compiler_params=pltpu.CompilerParams(use_tc_tiling_on_sc=False),
    )
    def gather_kernel(smpl_hbm, obj_hbm, sidx_hbm, oidx_hbm,
                      out_s_hbm, out_o_hbm,
                      sidx_vm, oidx_vm, srows_vm, orows_vm, sem):
        b = lax.axis_index("s") * _NC + lax.axis_index("c")
        pltpu.sync_copy(sidx_hbm, sidx_vm)
        pltpu.sync_copy(oidx_hbm.at[b], oidx_vm)

        copies = []
        for j in range(ns_chunks):
            copies.append(pltpu.async_copy(
                smpl_hbm.at[b].at[sidx_vm.at[j]],
                srows_vm.at[pl.ds(j * nc, nc)], sem))
        for j in range(no_chunks):
            copies.append(pltpu.async_copy(
                obj_hbm.at[b].at[oidx_vm.at[j]],
                orows_vm.at[pl.ds(j * nc, nc)], sem))
        for c in copies:
            c.wait()

        pltpu.sync_copy(srows_vm, out_s_hbm.at[b])
        pltpu.sync_copy(orows_vm, out_o_hbm.at[b])

    return gather_kernel(smpl_tab, obj_tab, sidx2, oidx3)


def _tc_loss_sum(rel3, smpl_rows, obj_g3, tile):
    B, A_S, L3 = rel3.shape
    nt = A_S // tile

    def body(rel_ref, smpl_ref, obj_ref, out_ref):
        step = pl.program_id(0) * nt + pl.program_id(1)
        rel = rel_ref[0]
        smpl = smpl_ref[0]
        obj = obj_ref[0]
        col = lax.broadcasted_iota(jnp.int32, (tile, 8), 1)
        smpl_aug = jnp.where(col == 3, 1.0, smpl)
        lane = lax.broadcasted_iota(jnp.int32, (8, L3), 1)
        row = lax.broadcasted_iota(jnp.int32, (8, L3), 0)
        lane_c = lane - (lane // 3) * 3
        m8 = jnp.where(row == 3, obj, jnp.where(row == lane_c, -1.0, 0.0))
        base = lax.dot_general(
            smpl_aug, m8, (((1,), (0,)), ((), ())),
            preferred_element_type=jnp.float32)
        absdiff = jnp.abs(base - rel)
        ones = jnp.ones((L3, 1), jnp.float32)
        psum = jnp.sum(jax.lax.dot_general(
            absdiff, ones, (((1,), (0,)), ((), ())),
            preferred_element_type=jnp.float32))

        @pl.when(step == 0)
        def _():
            out_ref[0, 0] = psum

        @pl.when(step != 0)
        def _():
            out_ref[0, 0] += psum

    return pl.pallas_call(
        body,
        grid=(B, nt),
        in_specs=[
            pl.BlockSpec((1, tile, L3), lambda b, i: (b, i, 0)),
            pl.BlockSpec((1, tile, _ROW), lambda b, i: (b, i, 0)),
            pl.BlockSpec((1, 1, L3), lambda b, i: (b, 0, 0)),
        ],
        out_specs=pl.BlockSpec(memory_space=pltpu.SMEM),
        out_shape=jax.ShapeDtypeStruct((1, 1), jnp.float32),
        compiler_params=pltpu.CompilerParams(
            dimension_semantics=("arbitrary", "arbitrary")),
    )(rel3, smpl_rows, obj_g3)


def kernel(smpl_v_orig, object_v_orig, rel_dist, smpl_anchor_indices,
           object_anchor_indices):
    B, n_smpl, _ = smpl_v_orig.shape
    n_obj = object_v_orig.shape[1]
    A_S = smpl_anchor_indices.shape[0]
    A_O = object_anchor_indices.shape[1]

    pad = ((0, 0), (0, 0), (0, _ROW - 3))
    smpl_tab = jnp.pad(smpl_v_orig, pad)
    obj_tab = jnp.pad(object_v_orig, pad)

    sidx2 = smpl_anchor_indices.astype(jnp.int32).reshape(
        A_S // _CHUNK, _CHUNK)
    oidx3 = object_anchor_indices.astype(jnp.int32).reshape(
        B, A_O // _CHUNK, _CHUNK)

    smpl_rows, obj_rows = _sc_gather(smpl_tab, obj_tab, sidx2, oidx3)

    rel3 = rel_dist.reshape(B, A_S, A_O * 3)
    obj_g3 = obj_rows[:, :, :3].reshape(B, 1, A_O * 3)

    total = _tc_loss_sum(rel3, smpl_rows, obj_g3, tile=1024)
    count = B * A_S * A_O * 3
    return (total / count).reshape(())
